# Initial kernel scaffold; baseline (speedup 1.0000x reference)
import functools

import jax
import jax.numpy as jnp
from jax import lax
from jax.experimental import pallas as pl
from jax.experimental.pallas import tpu as pltpu
from jax.experimental.pallas import tpu_sc as plsc

NC = 2
NS = 16
NW = NC * NS
L = 16


def _sc_mesh():
    return plsc.VectorSubcoreMesh(core_axis_name="c", subcore_axis_name="s")


def _make_degree_kernel(E, NP):
    EC = E // NW

    @functools.partial(
        pl.kernel,
        out_type=(
            jax.ShapeDtypeStruct((NW, NP), jnp.float32),
            jax.ShapeDtypeStruct((NW, NP), jnp.float32),
        ),
        mesh=_sc_mesh(),
        scratch_types=[
            pltpu.VMEM((EC,), jnp.int32),
            pltpu.VMEM((EC,), jnp.int32),
            pltpu.VMEM((NP,), jnp.float32),
            pltpu.VMEM((NP,), jnp.float32),
        ],
    )
    def deg_kernel(src_hbm, dst_hbm, osrc_hbm, odst_hbm, src_v, dst_v, hs_v, hd_v):
        cid = lax.axis_index("c")
        sid = lax.axis_index("s")
        wid = sid * NC + cid
        base = wid * EC
        pltpu.sync_copy(src_hbm.at[pl.ds(base, EC)], src_v)
        pltpu.sync_copy(dst_hbm.at[pl.ds(base, EC)], dst_v)

        zeros = jnp.zeros((L,), jnp.float32)

        def zbody(i, carry):
            hs_v[pl.ds(i * L, L)] = zeros
            hd_v[pl.ds(i * L, L)] = zeros
            return carry

        lax.fori_loop(0, NP // L, zbody, 0)

        ones = jnp.ones((L,), jnp.float32)

        def hbody(i, carry):
            plsc.addupdate_scatter(hs_v, [src_v[pl.ds(i * L, L)]], ones)
            plsc.addupdate_scatter(hd_v, [dst_v[pl.ds(i * L, L)]], ones)
            return carry

        lax.fori_loop(0, EC // L, hbody, 0)

        pltpu.sync_copy(hs_v, osrc_hbm.at[wid])
        pltpu.sync_copy(hd_v, odst_hbm.at[wid])

    return deg_kernel


def _make_scatter_kernel(E, NP, D, K):
    EC = E // NW
    G = EC // K
    RPT = NP // NS

    @functools.partial(
        pl.kernel,
        out_type=jax.ShapeDtypeStruct((NC, NP, D), jnp.float32),
        mesh=_sc_mesh(),
        scratch_types=[
            pltpu.VMEM((G, K), jnp.int32),
            pltpu.VMEM((G, K), jnp.int32),
            pltpu.VMEM((K, D), jnp.float32),
            pltpu.VMEM_SHARED((NP, D), jnp.float32),
            pltpu.SemaphoreType.DMA,
        ],
    )
    def scatter_kernel(hs_hbm, srcr_hbm, dstr_hbm, out_hbm,
                       si_v, di_v, rows_v, acc_sh, sem):
        cid = lax.axis_index("c")
        sid = lax.axis_index("s")
        wid = sid * NC + cid
        pltpu.sync_copy(srcr_hbm.at[wid], si_v)
        pltpu.sync_copy(dstr_hbm.at[wid], di_v)

        zeros = jnp.zeros((L,), jnp.float32)

        def zbody(r, carry):
            for k in range(D // L):
                rows_v[r, pl.ds(k * L, L)] = zeros
            return carry

        lax.fori_loop(0, K, zbody, 0)
        row0 = sid * RPT
        for j in range(RPT // K):
            pltpu.sync_copy(rows_v, acc_sh.at[pl.ds(row0 + j * K, K)])
        plsc.subcore_barrier()

        def ebody(g, carry):
            pltpu.async_copy(hs_hbm.at[si_v.at[g]], rows_v, sem).wait()
            pltpu.sync_copy(rows_v, acc_sh.at[di_v.at[g]], add=True)
            return carry

        lax.fori_loop(0, G, ebody, 0)
        plsc.subcore_barrier()

        for j in range(RPT // K):
            pltpu.sync_copy(acc_sh.at[pl.ds(row0 + j * K, K)], rows_v)
            pltpu.sync_copy(rows_v, out_hbm.at[cid, pl.ds(row0 + j * K, K)])

    return scatter_kernel


def _mm_body(x_ref, w_ref, degp_ref, hs_ref):
    deg = jnp.sum(degp_ref[...], axis=0)
    inv = lax.rsqrt(jnp.maximum(deg, 1.0))
    xs = x_ref[...] * inv[:, None]
    hs_ref[...] = jnp.dot(xs, w_ref[...],
                          preferred_element_type=jnp.float32,
                          precision=lax.Precision.HIGHEST)


def _fin_body(accp_ref, degp_ref, b_ref, out_ref):
    acc = accp_ref[0] + accp_ref[1]
    deg = jnp.sum(degp_ref[...], axis=0)
    inv = lax.rsqrt(jnp.maximum(deg, 1.0))
    out_ref[...] = acc * inv[:, None] + b_ref[...]


def kernel(x, edge_index, W, b):
    N, D = x.shape
    E = edge_index.shape[1]
    NP = ((N + 511) // 512) * 512
    K = 80
    BR = 2048

    src = edge_index[0]
    dst = edge_index[1]
    EC = E // NW
    G = EC // K
    src_r = src.reshape(NW, G, K)
    dst_r = dst.reshape(NW, G, K)

    deg_src_p, deg_dst_p = _make_degree_kernel(E, NP)(src, dst)

    x_pad = jnp.pad(x, ((0, NP - N), (0, 0)))
    hs = pl.pallas_call(
        _mm_body,
        grid=(NP // BR,),
        in_specs=[
            pl.BlockSpec((BR, D), lambda i: (i, 0)),
            pl.BlockSpec((D, D), lambda i: (0, 0)),
            pl.BlockSpec((NW, BR), lambda i: (0, i)),
        ],
        out_specs=pl.BlockSpec((BR, D), lambda i: (i, 0)),
        out_shape=jax.ShapeDtypeStruct((NP, D), jnp.float32),
    )(x_pad, W, deg_src_p)

    acc_p = _make_scatter_kernel(E, NP, D, K)(hs, src_r, dst_r)

    out_full = pl.pallas_call(
        _fin_body,
        grid=(NP // BR,),
        in_specs=[
            pl.BlockSpec((NC, BR, D), lambda i: (0, i, 0)),
            pl.BlockSpec((NW, BR), lambda i: (0, i)),
            pl.BlockSpec((D,), lambda i: (0,)),
        ],
        out_specs=pl.BlockSpec((BR, D), lambda i: (i, 0)),
        out_shape=jax.ShapeDtypeStruct((NP, D), jnp.float32),
    )(acc_p, deg_dst_p, b)

    return out_full[:N]


# speedup vs baseline: 25.8405x; 25.8405x over previous
import functools

import jax
import jax.numpy as jnp
from jax import lax
from jax.experimental import pallas as pl
from jax.experimental.pallas import tpu as pltpu
from jax.experimental.pallas import tpu_sc as plsc

NC = 2
NS = 16
NW = NC * NS
L = 16


def _sc_mesh():
    return plsc.VectorSubcoreMesh(core_axis_name="c", subcore_axis_name="s")


def _make_degree_kernel(E, NP):
    EC = E // NW

    @functools.partial(
        pl.kernel,
        out_type=(
            jax.ShapeDtypeStruct((NW, NP), jnp.float32),
            jax.ShapeDtypeStruct((NW, NP), jnp.float32),
        ),
        mesh=_sc_mesh(),
        scratch_types=[
            pltpu.VMEM((EC,), jnp.int32),
            pltpu.VMEM((EC,), jnp.int32),
            pltpu.VMEM((NP,), jnp.float32),
            pltpu.VMEM((NP,), jnp.float32),
        ],
        compiler_params=pltpu.CompilerParams(needs_layout_passes=False),
    )
    def deg_kernel(src_hbm, dst_hbm, osrc_hbm, odst_hbm, src_v, dst_v, hs_v, hd_v):
        cid = lax.axis_index("c")
        sid = lax.axis_index("s")
        wid = sid * NC + cid
        base = wid * EC
        pltpu.sync_copy(src_hbm.at[pl.ds(base, EC)], src_v)
        pltpu.sync_copy(dst_hbm.at[pl.ds(base, EC)], dst_v)

        zeros = jnp.zeros((L,), jnp.float32)

        def zbody(i, carry):
            hs_v[pl.ds(i * L, L)] = zeros
            hd_v[pl.ds(i * L, L)] = zeros
            return carry

        lax.fori_loop(0, NP // L, zbody, 0)

        ones = jnp.ones((L,), jnp.float32)

        def hbody(i, carry):
            plsc.addupdate_scatter(hs_v, [src_v[pl.ds(i * L, L)]], ones)
            plsc.addupdate_scatter(hd_v, [dst_v[pl.ds(i * L, L)]], ones)
            return carry

        lax.fori_loop(0, EC // L, hbody, 0)

        pltpu.sync_copy(hs_v, osrc_hbm.at[wid])
        pltpu.sync_copy(hd_v, odst_hbm.at[wid])

    return deg_kernel


def _make_scatter_kernel(E, NP, D, K):
    EC = E // NW
    G = EC // K
    RPT = NP // NS

    @functools.partial(
        pl.kernel,
        out_type=jax.ShapeDtypeStruct((NC, NP, D), jnp.float32),
        mesh=_sc_mesh(),
        scratch_types=[
            pltpu.VMEM((G, K), jnp.int32),
            pltpu.VMEM((G, K), jnp.int32),
            pltpu.VMEM((K, D), jnp.float32),
            pltpu.VMEM_SHARED((NP, D), jnp.float32),
            pltpu.SemaphoreType.DMA,
        ],
        compiler_params=pltpu.CompilerParams(needs_layout_passes=False),
    )
    def scatter_kernel(hs_hbm, srcr_hbm, dstr_hbm, out_hbm,
                       si_v, di_v, rows_v, acc_sh, sem):
        cid = lax.axis_index("c")
        sid = lax.axis_index("s")
        wid = sid * NC + cid
        pltpu.sync_copy(srcr_hbm.at[wid], si_v)
        pltpu.sync_copy(dstr_hbm.at[wid], di_v)

        zeros = jnp.zeros((L,), jnp.float32)

        def zbody(r, carry):
            for k in range(D // L):
                rows_v[r, pl.ds(k * L, L)] = zeros
            return carry

        lax.fori_loop(0, K, zbody, 0)
        row0 = sid * RPT
        for j in range(RPT // K):
            pltpu.sync_copy(rows_v, acc_sh.at[pl.ds(row0 + j * K, K)])
        plsc.subcore_barrier()

        def ebody(g, carry):
            pltpu.async_copy(hs_hbm.at[si_v.at[g]], rows_v, sem).wait()
            pltpu.sync_copy(rows_v, acc_sh.at[di_v.at[g]], add=True)
            return carry

        lax.fori_loop(0, G, ebody, 0)
        plsc.subcore_barrier()

        for j in range(RPT // K):
            pltpu.sync_copy(acc_sh.at[pl.ds(row0 + j * K, K)], rows_v)
            pltpu.sync_copy(rows_v, out_hbm.at[cid, pl.ds(row0 + j * K, K)])

    return scatter_kernel


def _mm_body(x_ref, w_ref, degp_ref, hs_ref):
    deg = jnp.sum(degp_ref[...], axis=0)
    inv = lax.rsqrt(jnp.maximum(deg, 1.0))
    xs = x_ref[...] * inv[:, None]
    hs_ref[...] = jnp.dot(xs, w_ref[...],
                          preferred_element_type=jnp.float32,
                          precision=lax.Precision.HIGHEST)


def _fin_body(accp_ref, degp_ref, b_ref, out_ref):
    acc = accp_ref[0] + accp_ref[1]
    deg = jnp.sum(degp_ref[...], axis=0)
    inv = lax.rsqrt(jnp.maximum(deg, 1.0))
    out_ref[...] = acc * inv[:, None] + b_ref[...]


def kernel(x, edge_index, W, b):
    N, D = x.shape
    E = edge_index.shape[1]
    NP = ((N + 511) // 512) * 512
    K = 80
    BR = 2048

    src = edge_index[0]
    dst = edge_index[1]
    EC = E // NW
    G = EC // K
    src_r = src.reshape(NW, G, K)
    dst_r = dst.reshape(NW, G, K)

    deg_src_p, deg_dst_p = _make_degree_kernel(E, NP)(src, dst)

    x_pad = jnp.pad(x, ((0, NP - N), (0, 0)))
    hs = pl.pallas_call(
        _mm_body,
        grid=(NP // BR,),
        in_specs=[
            pl.BlockSpec((BR, D), lambda i: (i, 0)),
            pl.BlockSpec((D, D), lambda i: (0, 0)),
            pl.BlockSpec((NW, BR), lambda i: (0, i)),
        ],
        out_specs=pl.BlockSpec((BR, D), lambda i: (i, 0)),
        out_shape=jax.ShapeDtypeStruct((NP, D), jnp.float32),
    )(x_pad, W, deg_src_p)

    acc_p = _make_scatter_kernel(E, NP, D, K)(hs, src_r, dst_r)

    out_full = pl.pallas_call(
        _fin_body,
        grid=(NP // BR,),
        in_specs=[
            pl.BlockSpec((NC, BR, D), lambda i: (0, i, 0)),
            pl.BlockSpec((NW, BR), lambda i: (0, i)),
            pl.BlockSpec((D,), lambda i: (0,)),
        ],
        out_specs=pl.BlockSpec((BR, D), lambda i: (i, 0)),
        out_shape=jax.ShapeDtypeStruct((NP, D), jnp.float32),
    )(acc_p, deg_dst_p, b)

    return out_full[:N]
